# Initial kernel scaffold; baseline (speedup 1.0000x reference)
import jax
import jax.numpy as jnp
from jax.experimental import pallas as pl


def kernel(f1, f2, f3, emb1_table, emb2_table, dense_w, dense_b):
    raise NotImplementedError("write your pallas kernel here")


# speedup vs baseline: 23.4048x; 23.4048x over previous
import functools

import jax
import jax.numpy as jnp
from jax import lax
from jax.experimental import pallas as pl
from jax.experimental.pallas import tpu as pltpu
from jax.experimental.pallas import tpu_sc as plsc

BATCH = 16384
SEQ = 20
DIM = 16
NC = 2
NS = 16
NW = NC * NS
BPW = BATCH // NW
CB = 4
NCHUNK = BPW // CB
IDX_PER_GATHER = CB * SEQ


def _sc_body(f1r, f2r, f3r, t1, t2, wall, out_hbm,
             idx1, idx2, idx3, wv, outv, accbuf, rows1, rows2, rows3,
             sem0, sem1):
    wid = lax.axis_index("s") * NC + lax.axis_index("c")

    pltpu.sync_copy(f1r.at[wid], idx1)
    pltpu.sync_copy(f2r.at[wid], idx2)
    pltpu.sync_copy(f3r.at[wid], idx3)
    pltpu.sync_copy(wall, wv)

    bufs = (rows1, rows2, rows3)
    idxs = (idx1, idx2, idx3)
    tabs = (t1, t1, t2)
    sems = (sem0, sem1)

    def fire(c, p):
        for tab, idx, buf in zip(tabs, idxs, bufs):
            pltpu.async_copy(tab.at[idx.at[c]], buf.at[p], sems[p])

    def drain(p):
        for buf in bufs:
            pltpu.make_async_copy(
                t1.at[pl.ds(0, IDX_PER_GATHER)], buf.at[p], sems[p]).wait()

    def compute(c, p):
        accs = [jnp.zeros((DIM,), jnp.float32) for _ in range(CB)]
        for fi, buf in enumerate(bufs):
            r = buf.at[p]
            for s in range(SEQ):
                w = wv[fi * SEQ + s]
                for i in range(CB):
                    accs[i] = accs[i] + r[i * SEQ + s] * w
        for i in range(CB):
            accbuf[pl.ds((c * CB + i) * DIM, DIM)] = accs[i]

    fire(0, 0)
    fire(1, 1)

    def step(k, carry):
        c0 = 2 * k
        drain(0)
        compute(c0, 0)

        @pl.when(c0 + 2 < NCHUNK)
        def _():
            fire(c0 + 2, 0)

        drain(1)
        compute(c0 + 1, 1)

        @pl.when(c0 + 3 < NCHUNK)
        def _():
            fire(c0 + 3, 1)

        return carry

    lax.fori_loop(0, NCHUNK // 2, step, 0)

    lanes = lax.iota(jnp.int32, 16)

    def hsum(v):
        for k in (8, 4, 2, 1):
            v = v + jnp.take(v, lanes ^ k)
        return v

    def fin(g, carry):
        base = g * 16
        out_vec = jnp.zeros((16,), jnp.float32)
        for i in range(16):
            v = accbuf[pl.ds((base + i) * DIM, DIM)]
            out_vec = jnp.where(lanes == i, hsum(v), out_vec)
        outv[pl.ds(base, 16)] = out_vec
        return carry

    lax.fori_loop(0, BPW // 16, fin, 0)

    pltpu.sync_copy(outv, out_hbm.at[pl.ds(wid * BPW, BPW)])


@functools.lru_cache(maxsize=1)
def _build_sc_kernel():
    return functools.partial(
        pl.kernel,
        out_type=jax.ShapeDtypeStruct((BATCH,), jnp.float32),
        mesh=plsc.VectorSubcoreMesh(core_axis_name="c", subcore_axis_name="s",
                                    num_cores=NC, num_subcores=NS),
        scratch_types=[
            pltpu.VMEM((NCHUNK, IDX_PER_GATHER), jnp.int32),
            pltpu.VMEM((NCHUNK, IDX_PER_GATHER), jnp.int32),
            pltpu.VMEM((NCHUNK, IDX_PER_GATHER), jnp.int32),
            pltpu.VMEM((3 * SEQ, DIM), jnp.float32),
            pltpu.VMEM((BPW,), jnp.float32),
            pltpu.VMEM((BPW * DIM,), jnp.float32),
            pltpu.VMEM((2, IDX_PER_GATHER, DIM), jnp.float32),
            pltpu.VMEM((2, IDX_PER_GATHER, DIM), jnp.float32),
            pltpu.VMEM((2, IDX_PER_GATHER, DIM), jnp.float32),
            pltpu.SemaphoreType.DMA,
            pltpu.SemaphoreType.DMA,
        ],
        compiler_params=pltpu.CompilerParams(use_tc_tiling_on_sc=False),
    )(_sc_body)


def kernel(f1, f2, f3, emb1_table, emb2_table, dense_w, dense_b):
    f1r = f1.reshape(NW, NCHUNK, IDX_PER_GATHER)
    f2r = f2.reshape(NW, NCHUNK, IDX_PER_GATHER)
    f3r = f3.reshape(NW, NCHUNK, IDX_PER_GATHER)
    wperm = dense_w.reshape(SEQ, 3, DIM).transpose(1, 0, 2).reshape(3 * SEQ, DIM)
    out = _build_sc_kernel()(f1r, f2r, f3r, emb1_table, emb2_table, wperm)
    return out.reshape(BATCH, 1) + dense_b


# speedup vs baseline: 29.1021x; 1.2434x over previous
import functools

import jax
import jax.numpy as jnp
from jax import lax
from jax.experimental import pallas as pl
from jax.experimental.pallas import tpu as pltpu
from jax.experimental.pallas import tpu_sc as plsc

BATCH = 16384
SEQ = 20
DIM = 16
VOCAB = 1000000
NC = 2
NS = 16
NW = NC * NS
BPW = BATCH // NW
CB = 4
NCHUNK = BPW // CB
IDX_PER_GATHER = CB * SEQ


def _sc_body(f1r, f2r, f3r, t1, t2, wall, out_hbm,
             idx1, idx2, idx3, wv, outv, accbuf, rows1, rows2, rows3,
             sem0, sem1):
    wid = lax.axis_index("s") * NC + lax.axis_index("c")

    pltpu.sync_copy(f1r.at[wid], idx1)
    pltpu.sync_copy(f2r.at[wid], idx2)
    pltpu.sync_copy(f3r.at[wid], idx3)
    pltpu.sync_copy(wall, wv)

    bufs = (rows1, rows2, rows3)
    idxs = (idx1, idx2, idx3)
    tabs = (t1, t1, t2)
    sems = (sem0, sem1)

    def fire(c, p):
        for tab, idx, buf in zip(tabs, idxs, bufs):
            pltpu.async_copy(tab.at[idx.at[c]], buf.at[p], sems[p])

    def drain(p):
        for buf in bufs:
            pltpu.make_async_copy(
                t1.at[pl.ds(0, IDX_PER_GATHER)], buf.at[p], sems[p]).wait()

    def compute(c, p):
        accs = [jnp.zeros((DIM,), jnp.float32) for _ in range(CB)]
        for fi, buf in enumerate(bufs):
            r = buf.at[p]
            for s in range(SEQ):
                w = wv[fi * SEQ + s]
                for i in range(CB):
                    accs[i] = accs[i] + r[i * SEQ + s] * w
        for i in range(CB):
            accbuf[pl.ds((c * CB + i) * DIM, DIM)] = accs[i]

    fire(0, 0)
    fire(1, 1)

    def step(k, carry):
        c0 = 2 * k
        drain(0)
        compute(c0, 0)

        @pl.when(c0 + 2 < NCHUNK)
        def _():
            fire(c0 + 2, 0)

        drain(1)
        compute(c0 + 1, 1)

        @pl.when(c0 + 3 < NCHUNK)
        def _():
            fire(c0 + 3, 1)

        return carry

    lax.fori_loop(0, NCHUNK // 2, step, 0)

    lanes = lax.iota(jnp.int32, 16)

    def hsum(v):
        for k in (8, 4, 2, 1):
            v = v + jnp.take(v, lanes ^ k)
        return v

    def fin(g, carry):
        base = g * 16
        out_vec = jnp.zeros((16,), jnp.float32)
        for i in range(16):
            v = accbuf[pl.ds((base + i) * DIM, DIM)]
            out_vec = jnp.where(lanes == i, hsum(v), out_vec)
        outv[pl.ds(base, 16)] = out_vec
        return carry

    lax.fori_loop(0, BPW // 16, fin, 0)

    pltpu.sync_copy(outv, out_hbm.at[pl.ds(wid * BPW, BPW)])


@functools.lru_cache(maxsize=1)
def _build_sc_kernel():
    return functools.partial(
        pl.kernel,
        out_type=jax.ShapeDtypeStruct((BATCH,), jnp.float32),
        mesh=plsc.VectorSubcoreMesh(core_axis_name="c", subcore_axis_name="s",
                                    num_cores=NC, num_subcores=NS),
        scratch_types=[
            pltpu.VMEM((NCHUNK, IDX_PER_GATHER), jnp.int32),
            pltpu.VMEM((NCHUNK, IDX_PER_GATHER), jnp.int32),
            pltpu.VMEM((NCHUNK, IDX_PER_GATHER), jnp.int32),
            pltpu.VMEM((3 * SEQ, DIM), jnp.float32),
            pltpu.VMEM((BPW,), jnp.float32),
            pltpu.VMEM((BPW * DIM,), jnp.float32),
            pltpu.VMEM((2, IDX_PER_GATHER, DIM), jnp.float32),
            pltpu.VMEM((2, IDX_PER_GATHER, DIM), jnp.float32),
            pltpu.VMEM((2, IDX_PER_GATHER, DIM), jnp.float32),
            pltpu.SemaphoreType.DMA,
            pltpu.SemaphoreType.DMA,
        ],
        compiler_params=pltpu.CompilerParams(use_tc_tiling_on_sc=False),
    )(_sc_body)


CBLK = 2048
NBLK = (VOCAB + CBLK - 1) // CBLK
MROWS = CBLK // 8
PADROWS = NBLK * MROWS


def _tc_transpose(t1t, t2t):
    def body(t1_ref, t2_ref, o1_ref, o2_ref):
        for t_ref, o_ref in ((t1_ref, o1_ref), (t2_ref, o2_ref)):
            y = t_ref[...].T
            o_ref[...] = jnp.concatenate(
                [y[j * MROWS:(j + 1) * MROWS, :] for j in range(8)], axis=1)

    return pl.pallas_call(
        body,
        grid=(NBLK,),
        in_specs=[pl.BlockSpec((16, CBLK), lambda i: (0, i)),
                  pl.BlockSpec((16, CBLK), lambda i: (0, i))],
        out_specs=[pl.BlockSpec((MROWS, 128), lambda i: (i, 0)),
                   pl.BlockSpec((MROWS, 128), lambda i: (i, 0))],
        out_shape=[jax.ShapeDtypeStruct((PADROWS, 128), jnp.float32),
                   jax.ShapeDtypeStruct((PADROWS, 128), jnp.float32)],
    )(t1t, t2t)


def _perm(f):
    return (256 * (f // 2048) + f % 256) * 8 + (f // 256) % 8


def kernel(f1, f2, f3, emb1_table, emb2_table, dense_w, dense_b):
    f1r = _perm(f1).reshape(NW, NCHUNK, IDX_PER_GATHER)
    f2r = _perm(f2).reshape(NW, NCHUNK, IDX_PER_GATHER)
    f3r = _perm(f3).reshape(NW, NCHUNK, IDX_PER_GATHER)
    wperm = dense_w.reshape(SEQ, 3, DIM).transpose(1, 0, 2).reshape(3 * SEQ, DIM)
    o1, o2 = _tc_transpose(emb1_table.T, emb2_table.T)
    t1lin = o1.reshape(PADROWS * 8, DIM)
    t2lin = o2.reshape(PADROWS * 8, DIM)
    out = _build_sc_kernel()(f1r, f2r, f3r, t1lin, t2lin, wperm)
    return out.reshape(BATCH, 1) + dense_b


# speedup vs baseline: 45.4335x; 1.5612x over previous
import functools

import jax
import jax.numpy as jnp
from jax import lax
from jax.experimental import pallas as pl
from jax.experimental.pallas import tpu as pltpu
from jax.experimental.pallas import tpu_sc as plsc

BATCH = 16384
SEQ = 20
DIM = 16
VOCAB = 1000000
NC = 2
NS = 16
NW = NC * NS
BPW = BATCH // NW
CB = 4
NCHUNK = BPW // CB
IDX_PER_GATHER = CB * SEQ


def _sc_body(f1r, f2r, f3r, t1, t2, wall, out_hbm,
             idx1, idx2, idx3, wv, outv, accbuf, rows1, rows2, rows3,
             sem0, sem1):
    wid = lax.axis_index("s") * NC + lax.axis_index("c")

    pltpu.sync_copy(f1r.at[wid], idx1)
    pltpu.sync_copy(f2r.at[wid], idx2)
    pltpu.sync_copy(f3r.at[wid], idx3)
    pltpu.sync_copy(wall, wv)

    bufs = (rows1, rows2, rows3)
    idxs = (idx1, idx2, idx3)
    tabs = (t1, t1, t2)
    sems = (sem0, sem1)

    def fire(c, p):
        for tab, idx, buf in zip(tabs, idxs, bufs):
            pltpu.async_copy(tab.at[idx.at[c]], buf.at[p], sems[p])

    def drain(p):
        for buf in bufs:
            pltpu.make_async_copy(
                t1.at[pl.ds(0, IDX_PER_GATHER)], buf.at[p], sems[p]).wait()

    def compute(c, p):
        accs = [jnp.zeros((DIM,), jnp.float32) for _ in range(CB)]
        for fi, buf in enumerate(bufs):
            r = buf.at[p]
            for s in range(SEQ):
                w = wv[fi * SEQ + s]
                for i in range(CB):
                    accs[i] = accs[i] + r[i * SEQ + s] * w
        for i in range(CB):
            accbuf[pl.ds((c * CB + i) * DIM, DIM)] = accs[i]

    fire(0, 0)
    fire(1, 1)

    def step(k, carry):
        c0 = 2 * k
        drain(0)
        compute(c0, 0)

        @pl.when(c0 + 2 < NCHUNK)
        def _():
            fire(c0 + 2, 0)

        drain(1)
        compute(c0 + 1, 1)

        @pl.when(c0 + 3 < NCHUNK)
        def _():
            fire(c0 + 3, 1)

        return carry

    lax.fori_loop(0, NCHUNK // 2, step, 0)

    lanes = lax.iota(jnp.int32, 16)

    def hsum(v):
        for k in (8, 4, 2, 1):
            v = v + jnp.take(v, lanes ^ k)
        return v

    def fin(g, carry):
        base = g * 16
        out_vec = jnp.zeros((16,), jnp.float32)
        for i in range(16):
            v = accbuf[pl.ds((base + i) * DIM, DIM)]
            out_vec = jnp.where(lanes == i, hsum(v), out_vec)
        outv[pl.ds(base, 16)] = out_vec
        return carry

    lax.fori_loop(0, BPW // 16, fin, 0)

    pltpu.sync_copy(outv, out_hbm.at[pl.ds(wid * BPW, BPW)])


@functools.lru_cache(maxsize=1)
def _build_sc_kernel():
    return functools.partial(
        pl.kernel,
        out_type=jax.ShapeDtypeStruct((BATCH,), jnp.float32),
        mesh=plsc.VectorSubcoreMesh(core_axis_name="c", subcore_axis_name="s",
                                    num_cores=NC, num_subcores=NS),
        scratch_types=[
            pltpu.VMEM((NCHUNK, IDX_PER_GATHER), jnp.int32),
            pltpu.VMEM((NCHUNK, IDX_PER_GATHER), jnp.int32),
            pltpu.VMEM((NCHUNK, IDX_PER_GATHER), jnp.int32),
            pltpu.VMEM((3 * SEQ, DIM), jnp.float32),
            pltpu.VMEM((BPW,), jnp.float32),
            pltpu.VMEM((BPW * DIM,), jnp.float32),
            pltpu.VMEM((2, IDX_PER_GATHER, DIM), jnp.float32),
            pltpu.VMEM((2, IDX_PER_GATHER, DIM), jnp.float32),
            pltpu.VMEM((2, IDX_PER_GATHER, DIM), jnp.float32),
            pltpu.SemaphoreType.DMA,
            pltpu.SemaphoreType.DMA,
        ],
        compiler_params=pltpu.CompilerParams(use_tc_tiling_on_sc=False),
    )(_sc_body)


CBLK = 8192
NBLK = (VOCAB + CBLK - 1) // CBLK
MROWS = CBLK // 8
PADROWS = NBLK * MROWS


def _tc_transpose(t1t, t2t):
    def body(t1_ref, t2_ref, o1_ref, o2_ref):
        lanes = lax.broadcasted_iota(jnp.int32, (16, 128), 1)
        dd = lax.broadcasted_iota(jnp.int32, (16, 128), 0)
        for t_ref, o_ref in ((t1_ref, o1_ref), (t2_ref, o2_ref)):
            x = t_ref[...]
            parts = []
            for j in range(8):
                ej = jnp.where(lanes == dd + 16 * j, 1.0, 0.0)
                xj = x[:, j * MROWS:(j + 1) * MROWS]
                parts.append(lax.dot_general(
                    xj, ej, (((0,), (0,)), ((), ())),
                    preferred_element_type=jnp.float32))
            while len(parts) > 1:
                parts = [a + b for a, b in zip(parts[::2], parts[1::2])]
            o_ref[...] = parts[0]

    return pl.pallas_call(
        body,
        grid=(NBLK,),
        in_specs=[pl.BlockSpec((16, CBLK), lambda i: (0, i)),
                  pl.BlockSpec((16, CBLK), lambda i: (0, i))],
        out_specs=[pl.BlockSpec((MROWS, 128), lambda i: (i, 0)),
                   pl.BlockSpec((MROWS, 128), lambda i: (i, 0))],
        out_shape=[jax.ShapeDtypeStruct((PADROWS, 128), jnp.float32),
                   jax.ShapeDtypeStruct((PADROWS, 128), jnp.float32)],
        compiler_params=pltpu.CompilerParams(fuse_transposed_lhs_in_matmul=True),
    )(t1t, t2t)


def _perm(f):
    return (MROWS * (f // CBLK) + f % MROWS) * 8 + (f // MROWS) % 8


def kernel(f1, f2, f3, emb1_table, emb2_table, dense_w, dense_b):
    f1r = _perm(f1).reshape(NW, NCHUNK, IDX_PER_GATHER)
    f2r = _perm(f2).reshape(NW, NCHUNK, IDX_PER_GATHER)
    f3r = _perm(f3).reshape(NW, NCHUNK, IDX_PER_GATHER)
    wperm = dense_w.reshape(SEQ, 3, DIM).transpose(1, 0, 2).reshape(3 * SEQ, DIM)
    o1, o2 = _tc_transpose(emb1_table.T, emb2_table.T)
    t1lin = o1.reshape(PADROWS * 8, DIM)
    t2lin = o2.reshape(PADROWS * 8, DIM)
    out = _build_sc_kernel()(f1r, f2r, f3r, t1lin, t2lin, wperm)
    return out.reshape(BATCH, 1) + dense_b


# speedup vs baseline: 67.9529x; 1.4957x over previous
import functools

import jax
import jax.numpy as jnp
from jax import lax
from jax.experimental import pallas as pl
from jax.experimental.pallas import tpu as pltpu
from jax.experimental.pallas import tpu_sc as plsc

BATCH = 16384
SEQ = 20
DIM = 16
VOCAB = 1000000
NC = 2
NS = 16
NW = NC * NS
BPW = BATCH // NW
CB = 4
NCHUNK = BPW // CB
IDX_PER_GATHER = CB * SEQ


def _sc_body(f1r, f2r, f3r, t1, t2, wall, out_hbm,
             idx1, idx2, idx3, wv, outv, accbuf, rows1, rows2, rows3,
             sem0, sem1):
    wid = lax.axis_index("s") * NC + lax.axis_index("c")

    pltpu.sync_copy(f1r.at[wid], idx1)
    pltpu.sync_copy(f2r.at[wid], idx2)
    pltpu.sync_copy(f3r.at[wid], idx3)
    pltpu.sync_copy(wall, wv)

    bufs = (rows1, rows2, rows3)
    idxs = (idx1, idx2, idx3)
    tabs = (t1, t1, t2)
    sems = (sem0, sem1)

    def fire(c, p):
        for tab, idx, buf in zip(tabs, idxs, bufs):
            pltpu.async_copy(tab.at[idx.at[c]], buf.at[p], sems[p])

    def drain(p):
        for buf in bufs:
            pltpu.make_async_copy(
                t1.at[pl.ds(0, IDX_PER_GATHER)], buf.at[p], sems[p]).wait()

    def compute(c, p):
        accs = [jnp.zeros((DIM,), jnp.float32) for _ in range(CB)]
        for fi, buf in enumerate(bufs):
            r = buf.at[p]
            for s in range(SEQ):
                w = wv[fi * SEQ + s]
                for i in range(CB):
                    accs[i] = accs[i] + r[i * SEQ + s] * w
        for i in range(CB):
            accbuf[pl.ds((c * CB + i) * DIM, DIM)] = accs[i]

    fire(0, 0)
    fire(1, 1)

    def step(k, carry):
        c0 = 2 * k
        drain(0)
        compute(c0, 0)

        @pl.when(c0 + 2 < NCHUNK)
        def _():
            fire(c0 + 2, 0)

        drain(1)
        compute(c0 + 1, 1)

        @pl.when(c0 + 3 < NCHUNK)
        def _():
            fire(c0 + 3, 1)

        return carry

    lax.fori_loop(0, NCHUNK // 2, step, 0)

    lanes = lax.iota(jnp.int32, 16)

    def hsum(v):
        for k in (8, 4, 2, 1):
            v = v + jnp.take(v, lanes ^ k)
        return v

    def fin(g, carry):
        base = g * 16
        out_vec = jnp.zeros((16,), jnp.float32)
        for i in range(16):
            v = accbuf[pl.ds((base + i) * DIM, DIM)]
            out_vec = jnp.where(lanes == i, hsum(v), out_vec)
        outv[pl.ds(base, 16)] = out_vec
        return carry

    lax.fori_loop(0, BPW // 16, fin, 0)

    pltpu.sync_copy(outv, out_hbm.at[pl.ds(wid * BPW, BPW)])


@functools.lru_cache(maxsize=1)
def _build_sc_kernel():
    return functools.partial(
        pl.kernel,
        out_type=jax.ShapeDtypeStruct((BATCH,), jnp.float32),
        mesh=plsc.VectorSubcoreMesh(core_axis_name="c", subcore_axis_name="s",
                                    num_cores=NC, num_subcores=NS),
        scratch_types=[
            pltpu.VMEM((NCHUNK, IDX_PER_GATHER), jnp.int32),
            pltpu.VMEM((NCHUNK, IDX_PER_GATHER), jnp.int32),
            pltpu.VMEM((NCHUNK, IDX_PER_GATHER), jnp.int32),
            pltpu.VMEM((3 * SEQ, DIM), jnp.float32),
            pltpu.VMEM((BPW,), jnp.float32),
            pltpu.VMEM((BPW * DIM,), jnp.float32),
            pltpu.VMEM((2, IDX_PER_GATHER, DIM), jnp.float32),
            pltpu.VMEM((2, IDX_PER_GATHER, DIM), jnp.float32),
            pltpu.VMEM((2, IDX_PER_GATHER, DIM), jnp.float32),
            pltpu.SemaphoreType.DMA,
            pltpu.SemaphoreType.DMA,
        ],
        compiler_params=pltpu.CompilerParams(use_tc_tiling_on_sc=False),
    )(_sc_body)


CBLK = 8192
NBLK = (VOCAB + CBLK - 1) // CBLK
MROWS = CBLK // 8
PADROWS = NBLK * MROWS


def _tc_transpose(t1t, t2t):
    def body(t1_ref, t2_ref, o1_ref, o2_ref):
        qv = lax.broadcasted_iota(jnp.int32, (128, 128), 0)
        cv = lax.broadcasted_iota(jnp.int32, (128, 128), 1)
        eb = jnp.where(cv == qv, 1.0, 0.0)
        for t_ref, o_ref in ((t1_ref, o1_ref), (t2_ref, o2_ref)):
            x = t_ref[...]
            xcat = jnp.concatenate(
                [x[:, j * MROWS:(j + 1) * MROWS] for j in range(8)], axis=0)
            o_ref[...] = lax.dot_general(
                xcat, eb, (((0,), (0,)), ((), ())),
                preferred_element_type=jnp.float32)

    return pl.pallas_call(
        body,
        grid=(NBLK,),
        in_specs=[pl.BlockSpec((16, CBLK), lambda i: (0, i)),
                  pl.BlockSpec((16, CBLK), lambda i: (0, i))],
        out_specs=[pl.BlockSpec((MROWS, 128), lambda i: (i, 0)),
                   pl.BlockSpec((MROWS, 128), lambda i: (i, 0))],
        out_shape=[jax.ShapeDtypeStruct((PADROWS, 128), jnp.float32),
                   jax.ShapeDtypeStruct((PADROWS, 128), jnp.float32)],
        compiler_params=pltpu.CompilerParams(fuse_transposed_lhs_in_matmul=True),
    )(t1t, t2t)


def _perm(f):
    return (MROWS * (f // CBLK) + f % MROWS) * 8 + (f // MROWS) % 8


def kernel(f1, f2, f3, emb1_table, emb2_table, dense_w, dense_b):
    f1r = _perm(f1).reshape(NW, NCHUNK, IDX_PER_GATHER)
    f2r = _perm(f2).reshape(NW, NCHUNK, IDX_PER_GATHER)
    f3r = _perm(f3).reshape(NW, NCHUNK, IDX_PER_GATHER)
    wperm = dense_w.reshape(SEQ, 3, DIM).transpose(1, 0, 2).reshape(3 * SEQ, DIM)
    o1, o2 = _tc_transpose(emb1_table.T, emb2_table.T)
    t1lin = o1.reshape(PADROWS * 8, DIM)
    t2lin = o2.reshape(PADROWS * 8, DIM)
    out = _build_sc_kernel()(f1r, f2r, f3r, t1lin, t2lin, wperm)
    return out.reshape(BATCH, 1) + dense_b
